# baseline (device time: 26658 ns/iter reference)
import jax
import jax.numpy as jnp
from jax import lax
from jax.experimental import pallas as pl
from jax.experimental.pallas import tpu as pltpu

N_RING = 8
N_SUB = 4


def kernel(ids, E):
    v_loc, d = E.shape
    t = ids.shape[0]
    blk = t // N_RING
    sub = blk // N_SUB

    x = lax.axis_index("x")
    y = lax.axis_index("y")
    z = lax.axis_index("z")
    b_own = x * 4 + z

    ids_blk = lax.dynamic_slice(ids, (b_own * blk,), (blk,))
    local = ids_blk - y * v_loc
    in_range = (local >= 0) & (local < v_loc)
    safe = jnp.where(in_range, local, 0).astype(jnp.int32)
    sel = jnp.broadcast_to(
        in_range.reshape(N_SUB, sub, 1), (N_SUB, sub, 128)
    ).astype(jnp.bfloat16)

    def body(safe_ref, sel_ref, e_ref, out_ref, gbuf, pbuf, ybuf, red,
             cwbuf, ccwbuf, gsem, ysend, yrecv, cwsend, cwrecv,
             ccwsend, ccwrecv):
        x = lax.axis_index("x")
        y = lax.axis_index("y")
        z = lax.axis_index("z")
        p = jnp.where(x == 0, z, 7 - z)

        def gather_chunk(s):
            copies = []
            for i in range(s * sub, (s + 1) * sub):
                c = pltpu.make_async_copy(
                    e_ref.at[pl.ds(safe_ref[i], 1), :],
                    gbuf.at[pl.ds(i, 1), :],
                    gsem.at[s],
                )
                c.start()
                copies.append(c)
            return copies

        def ring_coords(q):
            q = q % N_RING
            qx = jnp.where(q < 4, 0, 1)
            qz = jnp.where(q < 4, q, 7 - q)
            return qx, qz

        def blk_of(q):
            q = q % N_RING
            return jnp.where(q < 4, q, 11 - q)

        nx, nz = ring_coords(p + 1)
        px, pz = ring_coords(p - 1)
        ypeer = (x, 1 - y, z)
        nxt = (nx, y, nz)
        prv = (px, y, pz)

        barrier = pltpu.get_barrier_semaphore()
        for nbr in (ypeer, nxt, prv):
            pl.semaphore_signal(
                barrier, inc=1, device_id=nbr,
                device_id_type=pl.DeviceIdType.MESH,
            )
        gcopies = gather_chunk(0)
        pl.semaphore_wait(barrier, 3)

        yx = [
            pltpu.make_async_remote_copy(
                src_ref=pbuf.at[s], dst_ref=ybuf.at[s],
                send_sem=ysend.at[s], recv_sem=yrecv.at[s],
                device_id=ypeer, device_id_type=pl.DeviceIdType.MESH,
            )
            for s in range(N_SUB)
        ]
        cw = [
            [
                pltpu.make_async_remote_copy(
                    src_ref=(red if h == 0 else cwbuf.at[h - 1]).at[s],
                    dst_ref=cwbuf.at[h, s],
                    send_sem=cwsend.at[h, s], recv_sem=cwrecv.at[h, s],
                    device_id=nxt, device_id_type=pl.DeviceIdType.MESH,
                )
                for s in range(N_SUB)
            ]
            for h in range(4)
        ]
        ccw = [
            [
                pltpu.make_async_remote_copy(
                    src_ref=(red if h == 0 else ccwbuf.at[h - 1]).at[s],
                    dst_ref=ccwbuf.at[h, s],
                    send_sem=ccwsend.at[h, s], recv_sem=ccwrecv.at[h, s],
                    device_id=prv, device_id_type=pl.DeviceIdType.MESH,
                )
                for s in range(N_SUB)
            ]
            for h in range(3)
        ]

        for s in range(N_SUB):
            for c in gcopies:
                c.wait()
            pbuf[s] = gbuf[s * sub:(s + 1) * sub, :].astype(jnp.bfloat16)
            yx[s].start()
            if s < N_SUB - 1:
                gcopies = gather_chunk(s + 1)

        for s in range(N_SUB):
            yx[s].wait_recv()
            red[s] = jnp.where(
                sel_ref[s][:, 0:1] > 0, pbuf[s], ybuf[s]
            )
            cw[0][s].start()
            ccw[0][s].start()
            out_ref[pl.ds(blk_of(p) * blk + s * sub, sub), :] = (
                red[s].astype(jnp.float32)
            )

        for h in range(4):
            for s in range(N_SUB):
                cw[h][s].wait_recv()
                if h < 3:
                    cw[h + 1][s].start()
                    ccw[h][s].wait_recv()
                    if h < 2:
                        ccw[h + 1][s].start()
                out_ref[pl.ds(blk_of(p - h - 1) * blk + s * sub, sub), :] = (
                    cwbuf[h, s].astype(jnp.float32)
                )
                if h < 3:
                    out_ref[pl.ds(blk_of(p + h + 1) * blk + s * sub, sub), :] = (
                        ccwbuf[h, s].astype(jnp.float32)
                    )

        for r in yx:
            r.wait_send()
        for grp in cw + ccw:
            for r in grp:
                r.wait_send()

    return pl.pallas_call(
        body,
        out_shape=jax.ShapeDtypeStruct((t, d), jnp.float32),
        in_specs=[
            pl.BlockSpec(memory_space=pltpu.SMEM),
            pl.BlockSpec(memory_space=pltpu.VMEM),
            pl.BlockSpec(memory_space=pl.ANY),
        ],
        out_specs=pl.BlockSpec(memory_space=pltpu.VMEM),
        scratch_shapes=[
            pltpu.VMEM((blk, d), jnp.float32),
            pltpu.VMEM((N_SUB, sub, d), jnp.bfloat16),
            pltpu.VMEM((N_SUB, sub, d), jnp.bfloat16),
            pltpu.VMEM((N_SUB, sub, d), jnp.bfloat16),
            pltpu.VMEM((4, N_SUB, sub, d), jnp.bfloat16),
            pltpu.VMEM((3, N_SUB, sub, d), jnp.bfloat16),
            pltpu.SemaphoreType.DMA((N_SUB,)),
            pltpu.SemaphoreType.DMA((N_SUB,)),
            pltpu.SemaphoreType.DMA((N_SUB,)),
            pltpu.SemaphoreType.DMA((4, N_SUB)),
            pltpu.SemaphoreType.DMA((4, N_SUB)),
            pltpu.SemaphoreType.DMA((3, N_SUB)),
            pltpu.SemaphoreType.DMA((3, N_SUB)),
        ],
        compiler_params=pltpu.CompilerParams(collective_id=0),
    )(safe, sel, E)


# device time: 24558 ns/iter; 1.0855x vs baseline; 1.0855x over previous
import jax
import jax.numpy as jnp
from jax import lax
from jax.experimental import pallas as pl
from jax.experimental.pallas import tpu as pltpu

N_RING = 8
N_SUB = 4


def kernel(ids, E):
    v_loc, d = E.shape
    t = ids.shape[0]
    blk = t // N_RING
    sub = blk // N_SUB

    x = lax.axis_index("x")
    y = lax.axis_index("y")
    z = lax.axis_index("z")
    b_own = x * 4 + z

    ids_blk = lax.dynamic_slice(ids, (b_own * blk,), (blk,))
    local = ids_blk - y * v_loc
    in_range = (local >= 0) & (local < v_loc)
    safe = jnp.where(in_range, local, 0).astype(jnp.int32)
    sel = jnp.broadcast_to(
        in_range.reshape(N_SUB, sub, 1), (N_SUB, sub, 128)
    ).astype(jnp.bfloat16)

    def body(safe_ref, sel_ref, e_ref, out_ref, gbuf, pbuf, ybuf, red,
             cwbuf, ccwbuf, gsem, ysend, yrecv, cwsend, cwrecv,
             ccwsend, ccwrecv):
        x = lax.axis_index("x")
        y = lax.axis_index("y")
        z = lax.axis_index("z")
        p = jnp.where(x == 0, z, 7 - z)

        def gather_chunk(s):
            copies = []
            for i in range(s * sub, (s + 1) * sub):
                c = pltpu.make_async_copy(
                    e_ref.at[pl.ds(safe_ref[i], 1), :],
                    gbuf.at[pl.ds(i, 1), :],
                    gsem.at[s],
                )
                c.start()
                copies.append(c)
            return copies

        def ring_coords(q):
            q = q % N_RING
            qx = jnp.where(q < 4, 0, 1)
            qz = jnp.where(q < 4, q, 7 - q)
            return qx, qz

        def blk_of(q):
            q = q % N_RING
            return jnp.where(q < 4, q, 11 - q)

        nx, nz = ring_coords(p + 1)
        px, pz = ring_coords(p - 1)
        ypeer = (x, 1 - y, z)
        nxt = (nx, y, nz)
        prv = (px, y, pz)

        barrier = pltpu.get_barrier_semaphore()
        for nbr in (ypeer, nxt, prv):
            pl.semaphore_signal(
                barrier, inc=1, device_id=nbr,
                device_id_type=pl.DeviceIdType.MESH,
            )
        gcopies = [gather_chunk(s) for s in range(N_SUB)]
        pl.semaphore_wait(barrier, 3)

        yx = [
            pltpu.make_async_remote_copy(
                src_ref=pbuf.at[s], dst_ref=ybuf.at[s],
                send_sem=ysend.at[s], recv_sem=yrecv.at[s],
                device_id=ypeer, device_id_type=pl.DeviceIdType.MESH,
            )
            for s in range(N_SUB)
        ]
        cw = [
            [
                pltpu.make_async_remote_copy(
                    src_ref=(red if h == 0 else cwbuf.at[h - 1]).at[s],
                    dst_ref=cwbuf.at[h, s],
                    send_sem=cwsend.at[h, s], recv_sem=cwrecv.at[h, s],
                    device_id=nxt, device_id_type=pl.DeviceIdType.MESH,
                )
                for s in range(N_SUB)
            ]
            for h in range(4)
        ]
        ccw = [
            [
                pltpu.make_async_remote_copy(
                    src_ref=(red if h == 0 else ccwbuf.at[h - 1]).at[s],
                    dst_ref=ccwbuf.at[h, s],
                    send_sem=ccwsend.at[h, s], recv_sem=ccwrecv.at[h, s],
                    device_id=prv, device_id_type=pl.DeviceIdType.MESH,
                )
                for s in range(N_SUB)
            ]
            for h in range(3)
        ]

        for s in range(N_SUB):
            for c in gcopies[s]:
                c.wait()
            pbuf[s] = gbuf[s * sub:(s + 1) * sub, :].astype(jnp.bfloat16)
            yx[s].start()

        for s in range(N_SUB):
            yx[s].wait_recv()
            red[s] = jnp.where(
                sel_ref[s][:, 0:1] > 0, pbuf[s], ybuf[s]
            )
            cw[0][s].start()
            ccw[0][s].start()
            out_ref[pl.ds(blk_of(p) * blk + s * sub, sub), :] = (
                red[s].astype(jnp.float32)
            )

        for h in range(4):
            for s in range(N_SUB):
                cw[h][s].wait_recv()
                if h < 3:
                    cw[h + 1][s].start()
                    ccw[h][s].wait_recv()
                    if h < 2:
                        ccw[h + 1][s].start()
                out_ref[pl.ds(blk_of(p - h - 1) * blk + s * sub, sub), :] = (
                    cwbuf[h, s].astype(jnp.float32)
                )
                if h < 3:
                    out_ref[pl.ds(blk_of(p + h + 1) * blk + s * sub, sub), :] = (
                        ccwbuf[h, s].astype(jnp.float32)
                    )

        for r in yx:
            r.wait_send()
        for grp in cw + ccw:
            for r in grp:
                r.wait_send()

    return pl.pallas_call(
        body,
        out_shape=jax.ShapeDtypeStruct((t, d), jnp.float32),
        in_specs=[
            pl.BlockSpec(memory_space=pltpu.SMEM),
            pl.BlockSpec(memory_space=pltpu.VMEM),
            pl.BlockSpec(memory_space=pl.ANY),
        ],
        out_specs=pl.BlockSpec(memory_space=pltpu.VMEM),
        scratch_shapes=[
            pltpu.VMEM((blk, d), jnp.float32),
            pltpu.VMEM((N_SUB, sub, d), jnp.bfloat16),
            pltpu.VMEM((N_SUB, sub, d), jnp.bfloat16),
            pltpu.VMEM((N_SUB, sub, d), jnp.bfloat16),
            pltpu.VMEM((4, N_SUB, sub, d), jnp.bfloat16),
            pltpu.VMEM((3, N_SUB, sub, d), jnp.bfloat16),
            pltpu.SemaphoreType.DMA((N_SUB,)),
            pltpu.SemaphoreType.DMA((N_SUB,)),
            pltpu.SemaphoreType.DMA((N_SUB,)),
            pltpu.SemaphoreType.DMA((4, N_SUB)),
            pltpu.SemaphoreType.DMA((4, N_SUB)),
            pltpu.SemaphoreType.DMA((3, N_SUB)),
            pltpu.SemaphoreType.DMA((3, N_SUB)),
        ],
        compiler_params=pltpu.CompilerParams(collective_id=0),
    )(safe, sel, E)


# device time: 23892 ns/iter; 1.1158x vs baseline; 1.0279x over previous
import jax
import jax.numpy as jnp
from jax import lax
from jax.experimental import pallas as pl
from jax.experimental.pallas import tpu as pltpu

N_RING = 8
N_SUB = 4


def kernel(ids, E):
    v_loc, d = E.shape
    t = ids.shape[0]
    blk = t // N_RING
    sub = blk // N_SUB

    x = lax.axis_index("x")
    y = lax.axis_index("y")
    z = lax.axis_index("z")
    b_own = x * 4 + z

    ids_blk = lax.dynamic_slice(ids, (b_own * blk,), (blk,))
    local = ids_blk - y * v_loc
    in_range = (local >= 0) & (local < v_loc)
    safe = jnp.where(in_range, local, 0).astype(jnp.int32)
    sel = jnp.broadcast_to(
        in_range.reshape(N_SUB, sub, 1), (N_SUB, sub, 128)
    ).astype(jnp.bfloat16)

    def body(safe_ref, sel_ref, e_ref, out_ref, gbuf, pbuf, ybuf, red,
             cwbuf, ccwbuf, gsem, ysend, yrecv, cwsend, cwrecv,
             ccwsend, ccwrecv):
        x = lax.axis_index("x")
        y = lax.axis_index("y")
        z = lax.axis_index("z")
        p = jnp.where(x == 0, z, 7 - z)

        def gather_chunk(s):
            copies = []
            for i in range(s * sub, (s + 1) * sub):
                c = pltpu.make_async_copy(
                    e_ref.at[pl.ds(safe_ref[i], 1), :],
                    gbuf.at[pl.ds(i, 1), :],
                    gsem.at[s],
                )
                c.start()
                copies.append(c)
            return copies

        def ring_coords(q):
            q = q % N_RING
            qx = jnp.where(q < 4, 0, 1)
            qz = jnp.where(q < 4, q, 7 - q)
            return qx, qz

        def blk_of(q):
            q = q % N_RING
            return jnp.where(q < 4, q, 11 - q)

        nx, nz = ring_coords(p + 1)
        px, pz = ring_coords(p - 1)
        ypeer = (x, 1 - y, z)
        nxt = (nx, y, nz)
        prv = (px, y, pz)

        barrier = pltpu.get_barrier_semaphore()
        for nbr in (ypeer, nxt, prv):
            pl.semaphore_signal(
                barrier, inc=1, device_id=nbr,
                device_id_type=pl.DeviceIdType.MESH,
            )
        gcopies = [gather_chunk(s) for s in range(N_SUB)]
        pl.semaphore_wait(barrier, 3)

        yx = [
            pltpu.make_async_remote_copy(
                src_ref=pbuf.at[s], dst_ref=ybuf.at[s],
                send_sem=ysend.at[s], recv_sem=yrecv.at[s],
                device_id=ypeer, device_id_type=pl.DeviceIdType.MESH,
            )
            for s in range(N_SUB)
        ]
        cw = [
            [
                pltpu.make_async_remote_copy(
                    src_ref=(red if h == 0 else cwbuf.at[h - 1]).at[s],
                    dst_ref=cwbuf.at[h, s],
                    send_sem=cwsend.at[h, s], recv_sem=cwrecv.at[h, s],
                    device_id=nxt, device_id_type=pl.DeviceIdType.MESH,
                )
                for s in range(N_SUB)
            ]
            for h in range(4)
        ]
        ccw = [
            [
                pltpu.make_async_remote_copy(
                    src_ref=(red if h == 0 else ccwbuf.at[h - 1]).at[s],
                    dst_ref=ccwbuf.at[h, s],
                    send_sem=ccwsend.at[h, s], recv_sem=ccwrecv.at[h, s],
                    device_id=prv, device_id_type=pl.DeviceIdType.MESH,
                )
                for s in range(N_SUB)
            ]
            for h in range(3)
        ]

        for s in range(N_SUB):
            for c in gcopies[s]:
                c.wait()
            pbuf[s] = gbuf[s * sub:(s + 1) * sub, :].astype(jnp.bfloat16)
            yx[s].start()

        for s in range(N_SUB):
            yx[s].wait_recv()
            red[s] = jnp.where(
                sel_ref[s][:, 0:1] > 0, pbuf[s], ybuf[s]
            )
            cw[0][s].start()
            ccw[0][s].start()
            out_ref[pl.ds(blk_of(p) * blk + s * sub, sub), :] = red[s]

        for h in range(4):
            for s in range(N_SUB):
                cw[h][s].wait_recv()
                if h < 3:
                    cw[h + 1][s].start()
                    ccw[h][s].wait_recv()
                    if h < 2:
                        ccw[h + 1][s].start()
                out_ref[pl.ds(blk_of(p - h - 1) * blk + s * sub, sub), :] = (
                    cwbuf[h, s]
                )
                if h < 3:
                    out_ref[pl.ds(blk_of(p + h + 1) * blk + s * sub, sub), :] = (
                        ccwbuf[h, s]
                    )

        for r in yx:
            r.wait_send()
        for grp in cw + ccw:
            for r in grp:
                r.wait_send()

    return pl.pallas_call(
        body,
        out_shape=jax.ShapeDtypeStruct((t, d), jnp.bfloat16),
        in_specs=[
            pl.BlockSpec(memory_space=pltpu.SMEM),
            pl.BlockSpec(memory_space=pltpu.VMEM),
            pl.BlockSpec(memory_space=pl.ANY),
        ],
        out_specs=pl.BlockSpec(memory_space=pltpu.VMEM),
        scratch_shapes=[
            pltpu.VMEM((blk, d), jnp.float32),
            pltpu.VMEM((N_SUB, sub, d), jnp.bfloat16),
            pltpu.VMEM((N_SUB, sub, d), jnp.bfloat16),
            pltpu.VMEM((N_SUB, sub, d), jnp.bfloat16),
            pltpu.VMEM((4, N_SUB, sub, d), jnp.bfloat16),
            pltpu.VMEM((3, N_SUB, sub, d), jnp.bfloat16),
            pltpu.SemaphoreType.DMA((N_SUB,)),
            pltpu.SemaphoreType.DMA((N_SUB,)),
            pltpu.SemaphoreType.DMA((N_SUB,)),
            pltpu.SemaphoreType.DMA((4, N_SUB)),
            pltpu.SemaphoreType.DMA((4, N_SUB)),
            pltpu.SemaphoreType.DMA((3, N_SUB)),
            pltpu.SemaphoreType.DMA((3, N_SUB)),
        ],
        compiler_params=pltpu.CompilerParams(collective_id=0),
    )(safe, sel, E)
